# baseline (speedup 1.0000x reference)
import functools

import jax
import jax.numpy as jnp
import numpy as np
from jax import lax
from jax.experimental import pallas as pl
from jax.experimental.pallas import tpu as pltpu
from jax.experimental.pallas import tpu_sc as plsc

B = 16
TOK_LEN = 2048
N = B * TOK_LEN
SRC_LEN = 2048
POS_DIM = 64
ENC = 256
REL = 128
POS_VOCAB = 1000

NC = 2
NS = 16
NW = NC * NS
TOK_PER_W = N // NW
CHUNK = 128
NCHUNK = TOK_PER_W // CHUNK
LANES = 16

def _mm_body(x_ref, w2_ref, lut_ref, w1_ref, b_ref, y_ref, p_ref):
    y_ref[...] = lax.dot_general(
        x_ref[...], w2_ref[...], (((1,), (1,)), ((), ())),
        preferred_element_type=jnp.float32)

    @pl.when(pl.program_id(0) == 0)
    def _():
        p_ref[...] = lax.dot_general(
            lut_ref[...], w1_ref[...], (((1,), (1,)), ((), ())),
            preferred_element_type=jnp.float32) + b_ref[...]


def _sc_body(idx_hbm, pos_hbm, y_hbm, p_hbm, out_hbm,
             idx_v, pos_v, y_b, p_b, o_b, p_sh, sem_y, sem_p, sem_o):
    sid = lax.axis_index("s")
    wid = sid * NC + lax.axis_index("c")
    base = wid * TOK_PER_W

    @pl.when(sid == 0)
    def _():
        pltpu.sync_copy(p_hbm, p_sh)

    pltpu.sync_copy(idx_hbm.at[pl.ds(wid * NCHUNK, NCHUNK)], idx_v)
    pltpu.sync_copy(pos_hbm.at[pl.ds(wid * NCHUNK, NCHUNK)], pos_v)
    plsc.subcore_barrier()

    b_id = base // TOK_LEN
    off = b_id * SRC_LEN

    for c in range(NCHUNK):
        for v in range(CHUNK // LANES):
            s = idx_v[c, pl.ds(v * LANES, LANES)]
            idx_v[c, pl.ds(v * LANES, LANES)] = s + off

    def fire(c, s):
        return (pltpu.async_copy(y_hbm.at[idx_v.at[c]], y_b.at[s], sem_y.at[s]),
                pltpu.async_copy(p_sh.at[pos_v.at[c]], p_b.at[s], sem_p.at[s]))

    store_h = [None, None]
    g = fire(0, 0)
    for c in range(NCHUNK):
        s = c % 2
        if c + 1 < NCHUNK:
            g_next = fire(c + 1, 1 - s)
        g[0].wait()
        g[1].wait()
        if store_h[s] is not None:
            store_h[s].wait()

        def row(r, carry):
            for v in range(REL // LANES):
                yv = y_b[s, r, pl.ds(v * LANES, LANES)]
                pv = p_b[s, r, pl.ds(v * LANES, LANES)]
                o_b[s, r, pl.ds(v * LANES, LANES)] = jnp.maximum(
                    yv + pv, jnp.zeros_like(yv))
            return carry

        lax.fori_loop(0, CHUNK, row, 0)
        store_h[s] = pltpu.async_copy(
            o_b.at[s], out_hbm.at[pl.ds(base + c * CHUNK, CHUNK)], sem_o.at[s])
        if c + 1 < NCHUNK:
            g = g_next
    store_h[0].wait()
    store_h[1].wait()


@functools.lru_cache(maxsize=None)
def _make_sc_fuse():
    mesh = plsc.VectorSubcoreMesh(core_axis_name="c", subcore_axis_name="s")
    return pl.kernel(
        _sc_body,
        out_type=jax.ShapeDtypeStruct((N, REL), jnp.float32),
        mesh=mesh,
        scratch_types=[
            pltpu.VMEM((NCHUNK, CHUNK), jnp.int32),
            pltpu.VMEM((NCHUNK, CHUNK), jnp.int32),
            pltpu.VMEM((2, CHUNK, REL), jnp.float32),
            pltpu.VMEM((2, CHUNK, REL), jnp.float32),
            pltpu.VMEM((2, CHUNK, REL), jnp.float32),
            pltpu.VMEM_SHARED((POS_VOCAB, REL), jnp.float32),
            pltpu.SemaphoreType.DMA((2,)),
            pltpu.SemaphoreType.DMA((2,)),
            pltpu.SemaphoreType.DMA((2,)),
        ],
    )


def kernel(input_feats, lengths, index_local, src_enc_data, src_lengths,
           lut, W, b):
    pos = input_feats[:, 0]
    W1 = W[:, :POS_DIM]
    W2 = W[:, POS_DIM:]

    blk = 16384
    Y, P = pl.pallas_call(
        _mm_body,
        grid=(N // blk,),
        in_specs=[
            pl.BlockSpec((blk, ENC), lambda i: (i, 0)),
            pl.BlockSpec((REL, ENC), lambda i: (0, 0)),
            pl.BlockSpec((POS_VOCAB, POS_DIM), lambda i: (0, 0)),
            pl.BlockSpec((REL, POS_DIM), lambda i: (0, 0)),
            pl.BlockSpec((1, REL), lambda i: (0, 0)),
        ],
        out_specs=[
            pl.BlockSpec((blk, REL), lambda i: (i, 0)),
            pl.BlockSpec((POS_VOCAB, REL), lambda i: (0, 0)),
        ],
        out_shape=[
            jax.ShapeDtypeStruct((N, REL), jnp.float32),
            jax.ShapeDtypeStruct((POS_VOCAB, REL), jnp.float32),
        ],
    )(src_enc_data, W2, lut, W1, b.reshape(1, REL))

    idx2d = index_local.reshape(N // CHUNK, CHUNK)
    pos2d = pos.reshape(N // CHUNK, CHUNK)
    return _make_sc_fuse()(idx2d, pos2d, Y, P)


# speedup vs baseline: 1.0027x; 1.0027x over previous
import functools

import jax
import jax.numpy as jnp
import numpy as np
from jax import lax
from jax.experimental import pallas as pl
from jax.experimental.pallas import tpu as pltpu
from jax.experimental.pallas import tpu_sc as plsc

B = 16
TOK_LEN = 2048
N = B * TOK_LEN
SRC_LEN = 2048
POS_DIM = 64
ENC = 256
REL = 128
POS_VOCAB = 1000

NC = 2
NS = 16
NW = NC * NS
TOK_PER_W = N // NW
CHUNK = 128
NCHUNK = TOK_PER_W // CHUNK
LANES = 16

def _mm_body(x_ref, w2_ref, lut_ref, w1_ref, b_ref, y_ref, p_ref):
    y_ref[...] = lax.dot_general(
        x_ref[...], w2_ref[...], (((1,), (1,)), ((), ())),
        preferred_element_type=jnp.float32)

    @pl.when(pl.program_id(0) == 0)
    def _():
        p_ref[...] = lax.dot_general(
            lut_ref[...], w1_ref[...], (((1,), (1,)), ((), ())),
            preferred_element_type=jnp.float32) + b_ref[...]


def _sc_body(idx_hbm, pos_hbm, y_hbm, p_hbm, out_hbm,
             idx_v, pos_v, y_b, p_b, o_b, p_sh, sem_y, sem_p, sem_o):
    sid = lax.axis_index("s")
    wid = sid * NC + lax.axis_index("c")
    base = wid * TOK_PER_W

    @pl.when(sid == 0)
    def _():
        pltpu.sync_copy(p_hbm, p_sh)

    pltpu.sync_copy(idx_hbm.at[pl.ds(wid * NCHUNK, NCHUNK)], idx_v)
    pltpu.sync_copy(pos_hbm.at[pl.ds(wid * NCHUNK, NCHUNK)], pos_v)
    plsc.subcore_barrier()

    b_id = base // TOK_LEN
    off = b_id * SRC_LEN

    for c in range(NCHUNK):
        for v in range(CHUNK // LANES):
            s = idx_v[c, pl.ds(v * LANES, LANES)]
            idx_v[c, pl.ds(v * LANES, LANES)] = s + off

    def fire(c, s):
        return (pltpu.async_copy(y_hbm.at[idx_v.at[c]], y_b.at[s], sem_y.at[s]),
                pltpu.async_copy(p_sh.at[pos_v.at[c]], p_b.at[s], sem_p.at[s]))

    store_h = [None, None]
    g = fire(0, 0)
    for c in range(NCHUNK):
        s = c % 2
        if c + 1 < NCHUNK:
            g_next = fire(c + 1, 1 - s)
        g[0].wait()
        g[1].wait()
        if store_h[s] is not None:
            store_h[s].wait()

        def row(r, carry):
            for v in range(REL // LANES):
                yv = y_b[s, r, pl.ds(v * LANES, LANES)]
                pv = p_b[s, r, pl.ds(v * LANES, LANES)]
                o_b[s, r, pl.ds(v * LANES, LANES)] = jnp.maximum(
                    yv + pv, jnp.zeros_like(yv))
            return carry

        lax.fori_loop(0, CHUNK, row, 0)
        store_h[s] = pltpu.async_copy(
            o_b.at[s], out_hbm.at[pl.ds(base + c * CHUNK, CHUNK)], sem_o.at[s])
        if c + 1 < NCHUNK:
            g = g_next
    store_h[0].wait()
    store_h[1].wait()


@functools.lru_cache(maxsize=None)
def _make_sc_fuse():
    mesh = plsc.VectorSubcoreMesh(core_axis_name="c", subcore_axis_name="s")
    return pl.kernel(
        _sc_body,
        out_type=jax.ShapeDtypeStruct((N, REL), jnp.float32),
        mesh=mesh,
        scratch_types=[
            pltpu.VMEM((NCHUNK, CHUNK), jnp.int32),
            pltpu.VMEM((NCHUNK, CHUNK), jnp.int32),
            pltpu.VMEM((2, CHUNK, REL), jnp.float32),
            pltpu.VMEM((2, CHUNK, REL), jnp.float32),
            pltpu.VMEM((2, CHUNK, REL), jnp.float32),
            pltpu.VMEM_SHARED((POS_VOCAB, REL), jnp.float32),
            pltpu.SemaphoreType.DMA((2,)),
            pltpu.SemaphoreType.DMA((2,)),
            pltpu.SemaphoreType.DMA((2,)),
        ],
    )


def kernel(input_feats, lengths, index_local, src_enc_data, src_lengths,
           lut, W, b):
    pos = input_feats[:, 0]
    W1 = W[:, :POS_DIM]
    W2 = W[:, POS_DIM:]

    blk = 8192
    Y, P = pl.pallas_call(
        _mm_body,
        grid=(N // blk,),
        in_specs=[
            pl.BlockSpec((blk, ENC), lambda i: (i, 0)),
            pl.BlockSpec((REL, ENC), lambda i: (0, 0)),
            pl.BlockSpec((POS_VOCAB, POS_DIM), lambda i: (0, 0)),
            pl.BlockSpec((REL, POS_DIM), lambda i: (0, 0)),
            pl.BlockSpec((1, REL), lambda i: (0, 0)),
        ],
        out_specs=[
            pl.BlockSpec((blk, REL), lambda i: (i, 0)),
            pl.BlockSpec((POS_VOCAB, REL), lambda i: (0, 0)),
        ],
        out_shape=[
            jax.ShapeDtypeStruct((N, REL), jnp.float32),
            jax.ShapeDtypeStruct((POS_VOCAB, REL), jnp.float32),
        ],
    )(src_enc_data, W2, lut, W1, b.reshape(1, REL))

    idx2d = index_local.reshape(N // CHUNK, CHUNK)
    pos2d = pos.reshape(N // CHUNK, CHUNK)
    return _make_sc_fuse()(idx2d, pos2d, Y, P)


# speedup vs baseline: 1.0259x; 1.0231x over previous
import functools

import jax
import jax.numpy as jnp
import numpy as np
from jax import lax
from jax.experimental import pallas as pl
from jax.experimental.pallas import tpu as pltpu
from jax.experimental.pallas import tpu_sc as plsc

B = 16
TOK_LEN = 2048
N = B * TOK_LEN
SRC_LEN = 2048
POS_DIM = 64
ENC = 256
REL = 128
POS_VOCAB = 1000

NC = 2
NS = 16
NW = NC * NS
TOK_PER_W = N // NW
CHUNK = 128
NCHUNK = TOK_PER_W // CHUNK
LANES = 16

def _mm_body(x_ref, w_ref, lut_ref, b_ref, y_ref, p_ref):
    y_ref[...] = lax.dot_general(
        x_ref[...], w_ref[:, POS_DIM:], (((1,), (1,)), ((), ())),
        preferred_element_type=jnp.float32)

    @pl.when(pl.program_id(0) == 0)
    def _():
        p_ref[...] = lax.dot_general(
            lut_ref[...], w_ref[:, :POS_DIM], (((1,), (1,)), ((), ())),
            preferred_element_type=jnp.float32) + b_ref[...]


NFEAT = 5


def _sc_body(idx_hbm, pos_hbm, y_hbm, p_hbm, out_hbm,
             idx_v, pos_v, y_b, p_b, o_b, p_sh, sem_y, sem_p, sem_o):
    sid = lax.axis_index("s")
    wid = sid * NC + lax.axis_index("c")
    base = wid * TOK_PER_W

    @pl.when(sid == 0)
    def _():
        pltpu.sync_copy(p_hbm, p_sh)

    pltpu.sync_copy(idx_hbm.at[pl.ds(base, TOK_PER_W)], idx_v)
    pltpu.sync_copy(pos_hbm.at[pl.ds(base, TOK_PER_W)], pos_v)
    plsc.subcore_barrier()

    b_id = base // TOK_LEN
    off = b_id * SRC_LEN

    for c in range(NCHUNK):
        for v in range(CHUNK // LANES):
            t0 = c * CHUNK + v * LANES
            s = idx_v[pl.ds(t0, LANES)]
            idx_v[pl.ds(t0, LANES)] = s + off

    def fire(c, s):
        ic = idx_v.at[pl.ds(c * CHUNK, CHUNK)]
        pc = pos_v.at[pl.ds(c * CHUNK, CHUNK)]
        return (pltpu.async_copy(y_hbm.at[ic], y_b.at[s], sem_y.at[s]),
                pltpu.async_copy(p_sh.at[pc], p_b.at[s], sem_p.at[s]))

    store_h = [None, None]
    g = fire(0, 0)
    for c in range(NCHUNK):
        s = c % 2
        if c + 1 < NCHUNK:
            g_next = fire(c + 1, 1 - s)
        g[0].wait()
        g[1].wait()
        if store_h[s] is not None:
            store_h[s].wait()

        def row(r, carry):
            for v in range(REL // LANES):
                yv = y_b[s, r, pl.ds(v * LANES, LANES)]
                pv = p_b[s, r, pl.ds(v * LANES, LANES)]
                o_b[s, r, pl.ds(v * LANES, LANES)] = jnp.maximum(
                    yv + pv, jnp.zeros_like(yv))
            return carry

        lax.fori_loop(0, CHUNK, row, 0)
        store_h[s] = pltpu.async_copy(
            o_b.at[s], out_hbm.at[pl.ds(base + c * CHUNK, CHUNK)], sem_o.at[s])
        if c + 1 < NCHUNK:
            g = g_next
    store_h[0].wait()
    store_h[1].wait()


@functools.lru_cache(maxsize=None)
def _make_sc_fuse():
    mesh = plsc.VectorSubcoreMesh(core_axis_name="c", subcore_axis_name="s")
    return pl.kernel(
        _sc_body,
        out_type=jax.ShapeDtypeStruct((N, REL), jnp.float32),
        mesh=mesh,
        scratch_types=[
            pltpu.VMEM((TOK_PER_W,), jnp.int32),
            pltpu.VMEM((TOK_PER_W,), jnp.int32),
            pltpu.VMEM((2, CHUNK, REL), jnp.float32),
            pltpu.VMEM((2, CHUNK, REL), jnp.float32),
            pltpu.VMEM((2, CHUNK, REL), jnp.float32),
            pltpu.VMEM_SHARED((POS_VOCAB, REL), jnp.float32),
            pltpu.SemaphoreType.DMA((2,)),
            pltpu.SemaphoreType.DMA((2,)),
            pltpu.SemaphoreType.DMA((2,)),
        ],
    )


def kernel(input_feats, lengths, index_local, src_enc_data, src_lengths,
           lut, W, b):

    blk = 8192
    Y, P = pl.pallas_call(
        _mm_body,
        grid=(N // blk,),
        in_specs=[
            pl.BlockSpec((blk, ENC), lambda i: (i, 0)),
            pl.BlockSpec((REL, POS_DIM + ENC), lambda i: (0, 0)),
            pl.BlockSpec((POS_VOCAB, POS_DIM), lambda i: (0, 0)),
            pl.BlockSpec((1, REL), lambda i: (0, 0)),
        ],
        out_specs=[
            pl.BlockSpec((blk, REL), lambda i: (i, 0)),
            pl.BlockSpec((POS_VOCAB, REL), lambda i: (0, 0)),
        ],
        out_shape=[
            jax.ShapeDtypeStruct((N, REL), jnp.float32),
            jax.ShapeDtypeStruct((POS_VOCAB, REL), jnp.float32),
        ],
    )(src_enc_data, W, lut, b.reshape(1, REL))

    return _make_sc_fuse()(index_local, input_feats[:, 0], Y, P)
